# Initial kernel scaffold; baseline (speedup 1.0000x reference)
import jax
import jax.numpy as jnp
from jax.experimental import pallas as pl


def kernel(a_ECin, activity, W):
    raise NotImplementedError("write your pallas kernel here")


# speedup vs baseline: 1.0580x; 1.0580x over previous
import functools

import jax
import jax.numpy as jnp
from jax.experimental import pallas as pl
from jax.experimental.pallas import tpu as pltpu

N_IN = 4096
N_OUT = 16384
KTOP = max(1, int(0.01 * N_OUT))
TAU = 0.1
BC = 512
NB = N_OUT // BC


def _body(a_ref, w_ref, act_ref, out_ref, y_ref):
    i = pl.program_id(0)
    x = jnp.maximum(
        jnp.dot(a_ref[...], w_ref[...], preferred_element_type=jnp.float32), 0.0)
    y_ref[:, pl.ds(i * BC, BC)] = x / (x + 1.0)

    @pl.when(i == NB - 1)
    def _epilogue():
        y = y_ref[...]
        bits = jax.lax.bitcast_convert_type(y, jnp.int32)

        def step(_, carry):
            lo, hi = carry
            mid = (lo + hi) // 2
            cnt = jnp.sum((bits >= mid).astype(jnp.int32))
            big = cnt >= KTOP
            return (jnp.where(big, mid, lo), jnp.where(big, hi, mid))

        lo, _ = jax.lax.fori_loop(
            0, 31, step, (jnp.int32(0), jnp.int32(0x3F800000)))

        y_kwta = jnp.where(bits >= lo, y, 0.0)
        act = act_ref[...]
        out_ref[...] = act + TAU * (y_kwta - act)


@jax.jit
def kernel(a_ECin, activity, W):
    out = pl.pallas_call(
        _body,
        grid=(NB,),
        in_specs=[
            pl.BlockSpec((1, N_IN), lambda i: (0, 0)),
            pl.BlockSpec((N_IN, BC), lambda i: (0, i)),
            pl.BlockSpec((1, N_OUT), lambda i: (0, 0)),
        ],
        out_specs=pl.BlockSpec((1, N_OUT), lambda i: (0, 0)),
        out_shape=jax.ShapeDtypeStruct((1, N_OUT), jnp.float32),
        scratch_shapes=[pltpu.VMEM((1, N_OUT), jnp.float32)],
        compiler_params=pltpu.CompilerParams(
            dimension_semantics=("arbitrary",)),
    )(a_ECin.reshape(1, N_IN), W, activity.reshape(1, N_OUT))
    return out.reshape(N_OUT)


# speedup vs baseline: 1.0610x; 1.0028x over previous
import functools

import jax
import jax.numpy as jnp
from jax.experimental import pallas as pl
from jax.experimental.pallas import tpu as pltpu

N_IN = 4096
N_OUT = 16384
KTOP = max(1, int(0.01 * N_OUT))
TAU = 0.1
BC = 1024
NB = N_OUT // BC


def _body(a_ref, w_ref, act_ref, out_ref, y_ref):
    i = pl.program_id(0)
    x = jnp.maximum(
        jnp.dot(a_ref[...], w_ref[...], preferred_element_type=jnp.float32), 0.0)
    y_ref[:, pl.ds(i * BC, BC)] = x / (x + 1.0)

    @pl.when(i == NB - 1)
    def _epilogue():
        y = y_ref[...]
        bits = jax.lax.bitcast_convert_type(y, jnp.int32)

        def step(_, carry):
            lo, hi = carry
            mid = (lo + hi) // 2
            cnt = jnp.sum((bits >= mid).astype(jnp.int32))
            big = cnt >= KTOP
            return (jnp.where(big, mid, lo), jnp.where(big, hi, mid))

        lo, _ = jax.lax.fori_loop(
            0, 31, step, (jnp.int32(0), jnp.int32(0x3F800000)))

        y_kwta = jnp.where(bits >= lo, y, 0.0)
        act = act_ref[...]
        out_ref[...] = act + TAU * (y_kwta - act)


@jax.jit
def kernel(a_ECin, activity, W):
    out = pl.pallas_call(
        _body,
        grid=(NB,),
        in_specs=[
            pl.BlockSpec((1, N_IN), lambda i: (0, 0)),
            pl.BlockSpec((N_IN, BC), lambda i: (0, i)),
            pl.BlockSpec((1, N_OUT), lambda i: (0, 0)),
        ],
        out_specs=pl.BlockSpec((1, N_OUT), lambda i: (0, 0)),
        out_shape=jax.ShapeDtypeStruct((1, N_OUT), jnp.float32),
        scratch_shapes=[pltpu.VMEM((1, N_OUT), jnp.float32)],
        compiler_params=pltpu.CompilerParams(
            dimension_semantics=("arbitrary",)),
    )(a_ECin.reshape(1, N_IN), W, activity.reshape(1, N_OUT))
    return out.reshape(N_OUT)
